# Initial kernel scaffold; baseline (speedup 1.0000x reference)
import jax
import jax.numpy as jnp
from jax.experimental import pallas as pl


def kernel(unique_node_ids, unique_messages, timestamps, memory, last_update, W_ih, W_hh, b_ih, b_hh):
    raise NotImplementedError("write your pallas kernel here")


# speedup vs baseline: 2.6707x; 2.6707x over previous
import functools

import jax
import jax.numpy as jnp
from jax import lax
from jax.experimental import pallas as pl
from jax.experimental.pallas import tpu as pltpu
from jax.experimental.pallas import tpu_sc as plsc

NC = 2
NS = 16
NW = NC * NS


def _i32(x):
  return jnp.asarray(x, jnp.int32)


def _floop(n, body):
  lax.fori_loop(_i32(0), _i32(n), lambda i, c: (body(i), c)[1], _i32(0),
                unroll=False)


def _gather_body(ids2_hbm, mem_hbm, h_hbm, idx_v, rows16, sem):
  wid = lax.axis_index("c") * NS + lax.axis_index("s")
  rw = idx_v.shape[0]
  n = rw * 128
  pltpu.sync_copy(ids2_hbm.at[pl.ds(wid * rw, rw)], idx_v)

  def chunk(c):
    r, o = c // 8, c % 8
    idv = idx_v[r, pl.ds(o * 16, 16)]
    cps = []
    for j in range(16):
      cps.append(pltpu.async_copy(
          mem_hbm.at[pl.ds(idv[j], 1)], rows16.at[pl.ds(j, 1)], sem))
    for cp in cps:
      cp.wait()
    pltpu.sync_copy(rows16, h_hbm.at[pl.ds(wid * n + c * 16, 16)])

  _floop(n // 16, chunk)


def _make_gather(M, D, DM, B):
  rw = B // 128 // NW
  mesh = plsc.VectorSubcoreMesh(core_axis_name="c", subcore_axis_name="s")
  return pl.kernel(
      _gather_body,
      out_type=jax.ShapeDtypeStruct((B, D), jnp.float32),
      mesh=mesh,
      scratch_types=[
          pltpu.VMEM((rw, 128), jnp.int32),
          pltpu.VMEM((16, D), jnp.float32),
          pltpu.SemaphoreType.DMA,
      ],
  )


def _gru_body(x_ref, h_ref, wir, wiz, win, whr, whz, whn, br, bz, bni, bnh,
              out_ref):
  x = x_ref[...]
  h = h_ref[...]
  dot = functools.partial(
      lax.dot_general,
      dimension_numbers=(((1,), (0,)), ((), ())),
      preferred_element_type=jnp.float32)
  r = jax.nn.sigmoid(dot(x, wir[...]) + dot(h, whr[...]) + br[...])
  z = jax.nn.sigmoid(dot(x, wiz[...]) + dot(h, whz[...]) + bz[...])
  n = jnp.tanh(dot(x, win[...]) + bni[...] + r * (dot(h, whn[...]) + bnh[...]))
  res = (1.0 - z) * n + z * h
  out_ref[...] = jnp.concatenate([res, jnp.zeros_like(res)], axis=1)


def _make_gru(B, DM, D, blk):
  grid = (B // blk,)
  row_spec = lambda w: pl.BlockSpec((blk, w), lambda i: (i, _i32(0)))
  fix_spec = lambda a, b: pl.BlockSpec((a, b), lambda i: (_i32(0), _i32(0)))
  return pl.pallas_call(
      _gru_body,
      grid=grid,
      in_specs=[
          row_spec(DM),
          row_spec(D),
          fix_spec(DM, D), fix_spec(DM, D), fix_spec(DM, D),
          fix_spec(D, D), fix_spec(D, D), fix_spec(D, D),
          fix_spec(1, D), fix_spec(1, D), fix_spec(1, D), fix_spec(1, D),
      ],
      out_specs=row_spec(2 * D),
      out_shape=jax.ShapeDtypeStruct((B, 2 * D), jnp.float32),
  )


def _scatter_body(ids2_hbm, ts_hbm, newh_hbm, mem_ref, lu_ref,
                  ids_v, ts_v, tbl, l_id, l_b, rows16, rows16a, ts16, sem):
  rows = ids_v.shape[0]
  R = tbl.shape[0]
  D = mem_ref.shape[1]
  wid = lax.axis_index("c") * NS + lax.axis_index("s")
  base = wid * R
  iota = lax.iota(jnp.int32, 16)
  zeros16 = jnp.zeros((16,), jnp.int32)

  def zero(i):
    tbl[pl.ds(i * 16, 16)] = zeros16
  _floop(R // 16, zero)

  pltpu.sync_copy(ids2_hbm, ids_v)
  pltpu.sync_copy(ts_hbm, ts_v)

  def p1(r):
    for o in range(8):
      b0 = r * 128 + o * 16
      idv = ids_v[r, pl.ds(o * 16, 16)]
      local = idv - base
      m = (local >= 0) & (local < R)
      localc = jnp.where(m, local, 0)
      plsc.store_scatter(tbl, [localc], iota + (b0 + 1), mask=m)
  _floop(rows, p1)

  def p2(r, cnt):
    for o in range(8):
      b0 = r * 128 + o * 16
      idv = ids_v[r, pl.ds(o * 16, 16)]
      local = idv - base
      m = (local >= 0) & (local < R)
      localc = jnp.where(m, local, 0)
      g = plsc.load_gather(tbl, [localc], mask=m)
      keep = m & (g == iota + (b0 + 1))
      plsc.store_compressed(l_id.at[pl.ds(cnt, 16)], idv, mask=keep)
      plsc.store_compressed(l_b.at[pl.ds(cnt, 16)], iota + b0, mask=keep)
      cnt = cnt + plsc.all_reduce_population_count(keep)[0]
    return cnt

  cnt = lax.fori_loop(_i32(0), _i32(rows), p2, _i32(0))

  def chunk(c):
    off = c * 16
    valid = (off + iota) < cnt
    idv = l_id[pl.ds(off, 16)]
    bv = l_b[pl.ds(off, 16)]
    bvc = jnp.where(valid, bv, -1)
    idvc = jnp.where(valid, idv, -1)
    pltpu.async_copy(
        newh_hbm.at[plsc.Indices(bvc, ignored_value=-1)], rows16, sem).wait()
    ts16[pl.ds(0, 16)] = plsc.load_gather(ts_v, [jnp.where(valid, bv, 0)])
    for j in range(16):
      for q in range(D // 16):
        rows16a[j, pl.ds(q * 16, 16)] = rows16[j, pl.ds(q * 16, 16)]
    cps = []
    for j in range(16):
      cp = pltpu.make_async_copy(
          rows16a.at[pl.ds(j, 1)], mem_ref.at[pl.ds(idv[j], 1)], sem)
      cps.append(cp)
      @pl.when(off + j < cnt)
      def _start(cp=cp):
        cp.start()
    pltpu.async_copy(
        ts16, lu_ref.at[plsc.Indices(idvc, ignored_value=-1)], sem).wait()
    for j, cp in enumerate(cps):
      @pl.when(off + j < cnt)
      def _wait(cp=cp):
        cp.wait()

  _floop((cnt + 15) // 16, chunk)


def _make_scatter(M, D, DM, B):
  rows = B // 128
  R = ((M + NW - 1) // NW + 15) // 16 * 16
  mesh = plsc.VectorSubcoreMesh(core_axis_name="c", subcore_axis_name="s")
  return pl.kernel(
      _scatter_body,
      out_type=(),
      mesh=mesh,
      compiler_params=pltpu.CompilerParams(needs_layout_passes=False),
      scratch_types=[
          pltpu.VMEM((rows, 128), jnp.int32),
          pltpu.VMEM((B,), jnp.float32),
          pltpu.VMEM((R,), jnp.int32),
          pltpu.VMEM((B + 16,), jnp.int32),
          pltpu.VMEM((B + 16,), jnp.int32),
          pltpu.VMEM((16, DM), jnp.float32),
          pltpu.VMEM((16, D), jnp.float32),
          pltpu.VMEM((16,), jnp.float32),
          pltpu.SemaphoreType.DMA,
      ],
  )


def kernel(unique_node_ids, unique_messages, timestamps, memory, last_update,
           W_ih, W_hh, b_ih, b_hh):
  M, D = memory.shape
  B, DM = unique_messages.shape

  ids2 = unique_node_ids.astype(jnp.int32).reshape(B // 128, 128)
  W_ihT = W_ih.T.astype(jnp.float32)
  W_hhT = W_hh.T.astype(jnp.float32)
  wir, wiz, win = W_ihT[:, :D], W_ihT[:, D:2 * D], W_ihT[:, 2 * D:]
  whr, whz, whn = W_hhT[:, :D], W_hhT[:, D:2 * D], W_hhT[:, 2 * D:]
  br = (b_ih[:D] + b_hh[:D]).reshape(1, D)
  bz = (b_ih[D:2 * D] + b_hh[D:2 * D]).reshape(1, D)
  bni = b_ih[2 * D:].reshape(1, D)
  bnh = b_hh[2 * D:].reshape(1, D)

  h = _make_gather(M, D, DM, B)(ids2, memory)
  newh = _make_gru(B, DM, D, 512)(
      unique_messages, h, wir, wiz, win, whr, whz, whn, br, bz, bni, bnh)

  mem_ref = jax.new_ref(memory)
  lu_ref = jax.new_ref(last_update)
  _make_scatter(M, D, DM, B)(ids2, timestamps, newh, mem_ref, lu_ref)
  return mem_ref[...], lu_ref[...]
